# baseline (speedup 1.0000x reference)
import functools

import jax
import jax.numpy as jnp
from jax import lax
from jax.experimental import pallas as pl
from jax.experimental.pallas import tpu as pltpu
from jax.experimental.pallas import tpu_sc as plsc

DIM = 64
BATCH = 16384
HIST = 50
HHALF = HIST // 2
ROWS_C = BATCH * HHALF
NW = 32
R_PER_W = ROWS_C // NW
SLICE = 128
N_UNITS = R_PER_W // SLICE
NBUF = 10
LAG = 5
N_RND = N_UNITS // NBUF

_mesh = plsc.VectorSubcoreMesh(core_axis_name="c", subcore_axis_name="s")


@functools.partial(
    pl.kernel,
    mesh=_mesh,
    out_type=jax.ShapeDtypeStruct((ROWS_C, DIM), jnp.float32),
    scratch_types=[
        pltpu.VMEM((R_PER_W,), jnp.int32),
        pltpu.VMEM((NBUF, SLICE, DIM), jnp.float32),
    ] + [pltpu.SemaphoreType.DMA] * (2 * NBUF),
    compiler_params=pltpu.CompilerParams(use_tc_tiling_on_sc=False,
                                         needs_layout_passes=False),
)
def _gather_half(idx_hbm, table_hbm, out_hbm, idx_v, rows_v, *sems):
    gsem = sems[:NBUF]
    wsem = sems[NBUF:]
    wid = lax.axis_index("s") * 2 + lax.axis_index("c")
    base = wid * R_PER_W

    pltpu.sync_copy(idx_hbm.at[pl.ds(base, R_PER_W)], idx_v)

    def fire_gather(u, p):
        pltpu.async_copy(table_hbm.at[idx_v.at[pl.ds(u * SLICE, SLICE)]],
                         rows_v.at[p], gsem[p])

    def wait_gather(p):
        pltpu.make_async_copy(table_hbm.at[pl.ds(0, SLICE)],
                              rows_v.at[p], gsem[p]).wait()

    def wait_wb(p):
        pltpu.make_async_copy(rows_v.at[p], out_hbm.at[pl.ds(0, SLICE)],
                              wsem[p]).wait()

    def fire_wb(u, p):
        pltpu.async_copy(rows_v.at[p], out_hbm.at[pl.ds(base + u * SLICE,
                                                        SLICE)], wsem[p])

    def rnd(g, carry):
        for p in range(NBUF):
            j = g * NBUF + p

            @pl.when(g > 0)
            def _():
                wait_wb(p)

            fire_gather(j, p)

            u = j - LAG

            @pl.when(u >= 0)
            def _():
                q = (p - LAG) % NBUF
                wait_gather(q)
                fire_wb(u, q)

        return carry

    lax.fori_loop(0, N_RND, rnd, 0)

    for t in range(LAG):
        u = N_UNITS - LAG + t
        p = u % NBUF
        wait_gather(p)
        fire_wb(u, p)
    for p in range(NBUF):
        wait_wb(p)


def kernel(cell_indices, weight):
    idx = cell_indices.astype(jnp.int32)
    halves = [
        _gather_half(idx[:, h0:h0 + HHALF].reshape(ROWS_C),
                     weight).reshape(BATCH, HHALF, DIM)
        for h0 in (0, HHALF)
    ]
    return jnp.concatenate(halves, axis=1)


# speedup vs baseline: 1.0905x; 1.0905x over previous
import functools

import jax
import jax.numpy as jnp
from jax import lax
from jax.experimental import pallas as pl
from jax.experimental.pallas import tpu as pltpu
from jax.experimental.pallas import tpu_sc as plsc

DIM = 64
BATCH = 16384
HIST = 50
ROWS = BATCH * HIST
NW = 32
R_PER_W = ROWS // NW
SLICE = 128
N_UNITS = R_PER_W // SLICE
NBUF = 8
LAG = 4
N_RND = N_UNITS // NBUF

_mesh = plsc.VectorSubcoreMesh(core_axis_name="c", subcore_axis_name="s")


@functools.partial(
    pl.kernel,
    mesh=_mesh,
    out_type=jax.ShapeDtypeStruct((ROWS, DIM), jnp.float32),
    scratch_types=[
        pltpu.VMEM((R_PER_W,), jnp.int32),
        pltpu.VMEM((NBUF, SLICE, DIM), jnp.float32),
    ] + [pltpu.SemaphoreType.DMA] * (2 * NBUF),
    compiler_params=pltpu.CompilerParams(use_tc_tiling_on_sc=False,
                                         needs_layout_passes=False),
)
def _gather_all(idx_hbm, table_hbm, out_hbm, idx_v, rows_v, *sems):
    gsem = sems[:NBUF]
    wsem = sems[NBUF:]
    wid = lax.axis_index("s") * 2 + lax.axis_index("c")
    base = wid * R_PER_W

    pltpu.sync_copy(idx_hbm.at[pl.ds(base, R_PER_W)], idx_v)

    def fire_gather(u, p):
        pltpu.async_copy(table_hbm.at[idx_v.at[pl.ds(u * SLICE, SLICE)]],
                         rows_v.at[p], gsem[p])

    def wait_gather(p):
        pltpu.make_async_copy(table_hbm.at[pl.ds(0, SLICE)],
                              rows_v.at[p], gsem[p]).wait()

    def wait_wb(p):
        pltpu.make_async_copy(rows_v.at[p], out_hbm.at[pl.ds(0, SLICE)],
                              wsem[p]).wait()

    def fire_wb(u, p):
        pltpu.async_copy(rows_v.at[p], out_hbm.at[pl.ds(base + u * SLICE,
                                                        SLICE)], wsem[p])

    def rnd(g, carry):
        for p in range(NBUF):
            j = g * NBUF + p

            @pl.when(g > 0)
            def _():
                wait_wb(p)

            fire_gather(j, p)

            u = j - LAG

            @pl.when(u >= 0)
            def _():
                q = (p - LAG) % NBUF
                wait_gather(q)
                fire_wb(u, q)

        return carry

    lax.fori_loop(0, N_RND, rnd, 0)

    for t in range(LAG):
        u = N_UNITS - LAG + t
        p = u % NBUF
        wait_gather(p)
        fire_wb(u, p)
    for p in range(NBUF):
        wait_wb(p)


def kernel(cell_indices, weight):
    idx = cell_indices.astype(jnp.int32).reshape(ROWS)
    return _gather_all(idx, weight).reshape(BATCH, HIST, DIM)
